# baseline (speedup 1.0000x reference)
import functools

import jax
import jax.numpy as jnp
from jax import lax
from jax.experimental import pallas as pl
from jax.experimental.pallas import tpu as pltpu
from jax.experimental.pallas import tpu_sc as plsc

F = 26
V = 100000
D = 32
B = 16384
ND = 13

NC = 2
NS = 16
NW = NC * NS
CHUNK = 4096
NCH = B // CHUNK
FH = F // 2

BT = 1024
H1 = 512
H2 = 256


def _sc_gather_half(g, idx, deep_t, wide_t):
  mesh = plsc.VectorSubcoreMesh(core_axis_name="c", subcore_axis_name="s")

  @functools.partial(
      pl.kernel,
      out_type=(
          jax.ShapeDtypeStruct((FH * D, B), jnp.float32),
          jax.ShapeDtypeStruct((FH, B), jnp.float32),
      ),
      mesh=mesh,
      scratch_types=[
          pltpu.VMEM((1, V), jnp.float32),
          pltpu.VMEM((1, CHUNK), jnp.int32),
          pltpu.VMEM((1, CHUNK), jnp.int32),
          pltpu.VMEM((1, CHUNK), jnp.float32),
          pltpu.VMEM((1, CHUNK), jnp.float32),
          pltpu.SemaphoreType.DMA,
          pltpu.SemaphoreType.DMA,
          pltpu.SemaphoreType.DMA,
          pltpu.SemaphoreType.DMA,
          pltpu.SemaphoreType.DMA,
      ],
      compiler_params=pltpu.CompilerParams(use_tc_tiling_on_sc=True,
                                           needs_layout_passes=False),
      name=f"sc_gather_half{g}",
  )
  def k(idx_hbm, deep_hbm, wide_hbm, emb_out, wide_out,
        row_v, idx0_v, idx1_v, g0_v, g1_v,
        rsem, isem0, isem1, osem0, osem1):
    c = lax.axis_index("c")
    s = lax.axis_index("s")
    w = s * NC + c

    zero16 = jnp.zeros((16,), jnp.int32)
    idxb = (idx0_v, idx1_v)
    goutb = (g0_v, g1_v)
    isems = (isem0, isem1)
    osems = (osem0, osem1)

    def gather_chunk(idxc_v, gout_v):
      def gg(i, carry):
        for u in range(8):
          sl = pl.ds((i * 8 + u) * 16, 16)
          gout_v[0, sl] = plsc.load_gather(row_v, [zero16, idxc_v[0, sl]])
        return carry
      lax.fori_loop(0, CHUNK // (16 * 8), gg, 0)

    def row_pipeline(src_hbm, src_row, idx_row, out_hbm, out_row):
      def idx_slice(h):
        return idx_hbm.at[pl.ds(idx_row, 1), pl.ds(h * CHUNK, CHUNK)]

      def out_slice(h):
        return out_hbm.at[pl.ds(out_row, 1), pl.ds(h * CHUNK, CHUNK)]

      pltpu.async_copy(src_hbm.at[pl.ds(src_row, 1)], row_v, rsem)
      pltpu.async_copy(idx_slice(0), idxb[0], isems[0])
      pltpu.async_copy(idx_slice(1), idxb[1], isems[1])
      pltpu.make_async_copy(src_hbm.at[pl.ds(src_row, 1)], row_v,
                            rsem).wait()
      for h in range(NCH):
        b = h % 2
        pltpu.make_async_copy(idx_slice(h), idxb[b], isems[b]).wait()
        if h >= 2:
          pltpu.make_async_copy(goutb[b], out_slice(h - 2), osems[b]).wait()
        gather_chunk(idxb[b], goutb[b])
        if h + 2 < NCH:
          pltpu.async_copy(idx_slice(h + 2), idxb[b], isems[b])
        pltpu.async_copy(goutb[b], out_slice(h), osems[b])
      for h in (NCH - 2, NCH - 1):
        b = h % 2
        pltpu.make_async_copy(goutb[b], out_slice(h), osems[b]).wait()

    def deep_body(f, carry):
      row_pipeline(deep_hbm, (g * FH + f) * D + w, g * FH + f,
                   emb_out, f * D + w)
      return carry

    lax.fori_loop(0, FH, deep_body, 0)

    @pl.when(w < FH)
    def _wide():
      row_pipeline(wide_hbm, g * FH + w, g * FH + w, wide_out, w)

  return k(idx, deep_t, wide_t)


C0 = (((0,), (0,)), ((), ()))


def _tc_partial(emb_t, widev, dense_t, dwt, db, w1e, w1d, b1):
  def body(emb_ref, wv_ref, dense_ref, dwt_ref, db_ref, w1e_ref, w1d_ref,
           b1_ref, p_ref, ws_ref):
    bf = jnp.bfloat16
    dd = lax.dot_general(dense_ref[...], dwt_ref[...], C0,
                         preferred_element_type=jnp.float32) + db_ref[...]
    p = lax.dot_general(emb_ref[...].astype(bf), w1e_ref[...].astype(bf),
                        C0, preferred_element_type=jnp.float32)
    p = p + jnp.dot(dd, w1d_ref[...],
                    preferred_element_type=jnp.float32) + b1_ref[...]
    p_ref[...] = p.astype(bf)
    ws_ref[...] = lax.dot_general(wv_ref[...], jnp.ones((FH, 1),
                                                        jnp.float32), C0,
                                  preferred_element_type=jnp.float32)

  full = lambda a: pl.BlockSpec(a.shape, lambda i: (0,) * a.ndim)
  col_spec = lambda rows: pl.BlockSpec((rows, BT), lambda i: (0, i))
  return pl.pallas_call(
      body,
      grid=(B // BT,),
      in_specs=[
          col_spec(FH * D),
          col_spec(FH),
          col_spec(ND),
          full(dwt), full(db), full(w1e), full(w1d), full(b1),
      ],
      out_specs=(pl.BlockSpec((BT, H1), lambda i: (i, 0)),
                 pl.BlockSpec((BT, 1), lambda i: (i, 0))),
      out_shape=(jax.ShapeDtypeStruct((B, H1), jnp.bfloat16),
                 jax.ShapeDtypeStruct((B, 1), jnp.float32)),
  )(emb_t, widev, dense_t, dwt, db, w1e, w1d, b1)


def _tc_finish(emb_t, widev, p, ws1, dense_t, w1e, w2, b2, w3, b3,
               wwt, wb, bias):
  def body(emb_ref, wv_ref, p_ref, ws1_ref, dense_ref, w1e_ref, w2_ref,
           b2_ref, w3_ref, b3_ref, wwt_ref, wb_ref, bias_ref, out_ref):
    bf = jnp.bfloat16
    h1 = lax.dot_general(emb_ref[...].astype(bf), w1e_ref[...].astype(bf),
                         C0, preferred_element_type=jnp.float32)
    h1 = jnp.maximum(h1 + p_ref[...].astype(jnp.float32), 0.0)
    h2 = jnp.maximum(
        jnp.dot(h1.astype(bf), w2_ref[...].astype(bf),
                preferred_element_type=jnp.float32) + b2_ref[...], 0.0)
    h3 = jnp.maximum(
        jnp.dot(h2.astype(bf), w3_ref[...].astype(bf),
                preferred_element_type=jnp.float32) + b3_ref[...], 0.0)
    wd = lax.dot_general(dense_ref[...], wwt_ref[...], C0,
                         preferred_element_type=jnp.float32) + wb_ref[...]
    ws = lax.dot_general(wv_ref[...], jnp.ones((FH, 1), jnp.float32), C0,
                         preferred_element_type=jnp.float32)
    out_ref[...] = bias_ref[...] + ws1_ref[...] + ws + wd + h3

  full = lambda a: pl.BlockSpec(a.shape, lambda i: (0,) * a.ndim)
  col_spec = lambda rows: pl.BlockSpec((rows, BT), lambda i: (0, i))
  row_spec = lambda cols: pl.BlockSpec((BT, cols), lambda i: (i, 0))
  return pl.pallas_call(
      body,
      grid=(B // BT,),
      in_specs=[
          col_spec(FH * D),
          col_spec(FH),
          row_spec(H1),
          row_spec(1),
          col_spec(ND),
          full(w1e), full(w2), full(b2), full(w3), full(b3),
          full(wwt), full(wb), full(bias),
      ],
      out_specs=row_spec(1),
      out_shape=jax.ShapeDtypeStruct((B, 1), jnp.float32),
  )(emb_t, widev, p, ws1, dense_t, w1e, w2, b2, w3, b3, wwt, wb, bias)


def kernel(sparse_features, dense_features, wide_emb, wide_w, wide_b,
           deep_emb, deep_w, deep_b, W1, b1, W2, b2, W3, b3, bias):
  deep_t = deep_emb.transpose(0, 2, 1).reshape(F * D, V)
  wide_t = wide_emb.reshape(F, V)
  dense_t = dense_features.T

  emb0, widev0 = _sc_gather_half(0, sparse_features, deep_t, wide_t)
  emb1, widev1 = _sc_gather_half(1, sparse_features, deep_t, wide_t)

  p, ws1 = _tc_partial(
      emb0, widev0, dense_t,
      deep_w.T,
      deep_b.reshape(1, D),
      W1[:, D:D + FH * D].T,
      W1[:, :D].T,
      b1.reshape(1, H1),
  )
  return _tc_finish(
      emb1, widev1, p, ws1, dense_t,
      W1[:, D + FH * D:].T,
      W2.T, b2.reshape(1, H2),
      W3.T, b3.reshape(1, 1),
      wide_w.T, wide_b.reshape(1, 1),
      bias,
  )


# speedup vs baseline: 1.0278x; 1.0278x over previous
import functools

import jax
import jax.numpy as jnp
from jax import lax
from jax.experimental import pallas as pl
from jax.experimental.pallas import tpu as pltpu
from jax.experimental.pallas import tpu_sc as plsc

F = 26
V = 100000
D = 32
B = 16384
ND = 13

NC = 2
NS = 16
NW = NC * NS
CHUNK = 4096
NCH = B // CHUNK

BT = 1024


def _sc_gather(idx, deep_t, wide_t):
  mesh = plsc.VectorSubcoreMesh(core_axis_name="c", subcore_axis_name="s")

  @functools.partial(
      pl.kernel,
      out_type=(
          jax.ShapeDtypeStruct((F * D, B), jnp.float32),
          jax.ShapeDtypeStruct((F, B), jnp.float32),
      ),
      mesh=mesh,
      scratch_types=[
          pltpu.VMEM((1, V), jnp.float32),
          pltpu.VMEM((1, CHUNK), jnp.int32),
          pltpu.VMEM((1, CHUNK), jnp.int32),
          pltpu.VMEM((1, CHUNK), jnp.float32),
          pltpu.VMEM((1, CHUNK), jnp.float32),
          pltpu.SemaphoreType.DMA,
          pltpu.SemaphoreType.DMA,
          pltpu.SemaphoreType.DMA,
          pltpu.SemaphoreType.DMA,
          pltpu.SemaphoreType.DMA,
          pltpu.SemaphoreType.DMA,
      ],
      compiler_params=pltpu.CompilerParams(use_tc_tiling_on_sc=True,
                                           needs_layout_passes=False),
  )
  def k(idx_hbm, deep_hbm, wide_hbm, emb_out, wide_out,
        row_v, idx0_v, idx1_v, g0_v, g1_v,
        rsem, rsem2, isem0, isem1, osem0, osem1):
    c = lax.axis_index("c")
    s = lax.axis_index("s")
    w = s * NC + c

    zero16 = jnp.zeros((16,), jnp.int32)
    idxb = (idx0_v, idx1_v)
    goutb = (g0_v, g1_v)
    isems = (isem0, isem1)
    osems = (osem0, osem1)

    def gather_chunk(idxc_v, gout_v):
      def g(i, carry):
        for u in range(16):
          sl = pl.ds((i * 16 + u) * 16, 16)
          gout_v[0, sl] = plsc.load_gather(row_v, [zero16, idxc_v[0, sl]])
        return carry
      lax.fori_loop(0, CHUNK // (16 * 16), g, 0)

    def row_pipeline(src_hbm, src_row, idx_row, out_hbm, out_row):
      def idx_slice(h):
        return idx_hbm.at[pl.ds(idx_row, 1), pl.ds(h * CHUNK, CHUNK)]

      def out_slice(h):
        return out_hbm.at[pl.ds(out_row, 1), pl.ds(h * CHUNK, CHUNK)]

      VH = 50048
      row_lo_src = src_hbm.at[pl.ds(src_row, 1), pl.ds(0, VH)]
      row_hi_src = src_hbm.at[pl.ds(src_row, 1), pl.ds(VH, V - VH)]
      row_lo_dst = row_v.at[:, pl.ds(0, VH)]
      row_hi_dst = row_v.at[:, pl.ds(VH, V - VH)]
      pltpu.async_copy(row_lo_src, row_lo_dst, rsem)
      pltpu.async_copy(row_hi_src, row_hi_dst, rsem2)
      pltpu.async_copy(idx_slice(0), idxb[0], isems[0])
      pltpu.async_copy(idx_slice(1), idxb[1], isems[1])
      pltpu.make_async_copy(row_lo_src, row_lo_dst, rsem).wait()
      pltpu.make_async_copy(row_hi_src, row_hi_dst, rsem2).wait()
      for h in range(NCH):
        b = h % 2
        pltpu.make_async_copy(idx_slice(h), idxb[b], isems[b]).wait()
        if h >= 2:
          pltpu.make_async_copy(goutb[b], out_slice(h - 2), osems[b]).wait()
        gather_chunk(idxb[b], goutb[b])
        if h + 2 < NCH:
          pltpu.async_copy(idx_slice(h + 2), idxb[b], isems[b])
        pltpu.async_copy(goutb[b], out_slice(h), osems[b])
      for h in (NCH - 2, NCH - 1):
        b = h % 2
        pltpu.make_async_copy(goutb[b], out_slice(h), osems[b]).wait()

    def deep_body(f, carry):
      row_pipeline(deep_hbm, f * D + w, f, emb_out, f * D + w)
      return carry

    lax.fori_loop(0, F, deep_body, 0)

    @pl.when(w < F)
    def _wide():
      row_pipeline(wide_hbm, w, w, wide_out, w)

  return k(idx, deep_t, wide_t)


def _tc_mlp(emb_t, dense_t, wide2, dwt, db, w1e, w1d, b1, w2, b2, w3, b3,
            wwt, wb, bias):
  c0 = (((0,), (0,)), ((), ()))

  def body(emb_ref, dense_ref, ws_ref, dwt_ref, db_ref, w1e_ref, w1d_ref,
           b1_ref, w2_ref, b2_ref, w3_ref, b3_ref, wwt_ref, wb_ref,
           bias_ref, out_ref):
    bf = jnp.bfloat16
    dense_blk = dense_ref[...]
    dd = lax.dot_general(dense_blk, dwt_ref[...], c0,
                         preferred_element_type=jnp.float32) + db_ref[...]
    h1 = lax.dot_general(emb_ref[...].astype(bf), w1e_ref[...].astype(bf),
                         c0, preferred_element_type=jnp.float32)
    h1 = h1 + jnp.dot(dd, w1d_ref[...],
                      preferred_element_type=jnp.float32) + b1_ref[...]
    h1 = jnp.maximum(h1, 0.0)
    h2 = jnp.maximum(
        jnp.dot(h1.astype(bf), w2_ref[...].astype(bf),
                preferred_element_type=jnp.float32)
        + b2_ref[...], 0.0)
    h3 = jnp.maximum(
        jnp.dot(h2.astype(bf), w3_ref[...].astype(bf),
                preferred_element_type=jnp.float32)
        + b3_ref[...], 0.0)
    wd = lax.dot_general(dense_blk, wwt_ref[...], c0,
                         preferred_element_type=jnp.float32) + wb_ref[...]
    ws = lax.dot_general(ws_ref[...], jnp.ones((F, 1), jnp.float32), c0,
                         preferred_element_type=jnp.float32)
    out_ref[...] = bias_ref[...] + ws + wd + h3

  full = lambda a: pl.BlockSpec(a.shape, lambda i: (0,) * a.ndim)
  col_spec = lambda rows: pl.BlockSpec((rows, BT), lambda i: (0, i))
  return pl.pallas_call(
      body,
      grid=(B // BT,),
      in_specs=[
          col_spec(F * D),
          col_spec(ND),
          col_spec(F),
          full(dwt), full(db), full(w1e), full(w1d), full(b1),
          full(w2), full(b2), full(w3), full(b3),
          full(wwt), full(wb), full(bias),
      ],
      out_specs=pl.BlockSpec((BT, 1), lambda i: (i, 0)),
      out_shape=jax.ShapeDtypeStruct((B, 1), jnp.float32),
  )(emb_t, dense_t, wide2, dwt, db, w1e, w1d, b1, w2, b2, w3, b3,
    wwt, wb, bias)


def kernel(sparse_features, dense_features, wide_emb, wide_w, wide_b,
           deep_emb, deep_w, deep_b, W1, b1, W2, b2, W3, b3, bias):
  deep_t = deep_emb.transpose(0, 2, 1).reshape(F * D, V)
  wide_t = wide_emb.reshape(F, V)

  emb_t, wide2 = _sc_gather(sparse_features, deep_t, wide_t)

  return _tc_mlp(
      emb_t,
      dense_features.T,
      wide2,
      deep_w.T,
      deep_b.reshape(1, D),
      W1[:, D:].T,
      W1[:, :D].T,
      b1.reshape(1, 512),
      W2.T,
      b2.reshape(1, 256),
      W3.T,
      b3.reshape(1, 1),
      wide_w.T,
      wide_b.reshape(1, 1),
      bias,
  )


# speedup vs baseline: 1.0648x; 1.0360x over previous
import functools

import jax
import jax.numpy as jnp
from jax import lax
from jax.experimental import pallas as pl
from jax.experimental.pallas import tpu as pltpu
from jax.experimental.pallas import tpu_sc as plsc

F = 26
V = 100000
D = 32
B = 16384
ND = 13

NC = 2
NS = 16
NW = NC * NS
CHUNK = 4096
NCH = B // CHUNK

BT = 1024


def _sc_gather(idx, deep_t, wide_t):
  mesh = plsc.VectorSubcoreMesh(core_axis_name="c", subcore_axis_name="s")

  @functools.partial(
      pl.kernel,
      out_type=(
          jax.ShapeDtypeStruct((F * D, B), jnp.float32),
          jax.ShapeDtypeStruct((F, B), jnp.float32),
      ),
      mesh=mesh,
      scratch_types=[
          pltpu.VMEM((1, V), jnp.float32),
          pltpu.VMEM((1, CHUNK), jnp.int32),
          pltpu.VMEM((1, CHUNK), jnp.int32),
          pltpu.VMEM((1, CHUNK), jnp.float32),
          pltpu.VMEM((1, CHUNK), jnp.float32),
          pltpu.SemaphoreType.DMA,
          pltpu.SemaphoreType.DMA,
          pltpu.SemaphoreType.DMA,
          pltpu.SemaphoreType.DMA,
          pltpu.SemaphoreType.DMA,
      ],
      compiler_params=pltpu.CompilerParams(use_tc_tiling_on_sc=True,
                                           needs_layout_passes=False),
  )
  def k(idx_hbm, deep_hbm, wide_hbm, emb_out, wide_out,
        row_v, idx0_v, idx1_v, g0_v, g1_v,
        rsem, isem0, isem1, osem0, osem1):
    c = lax.axis_index("c")
    s = lax.axis_index("s")
    w = s * NC + c

    zero16 = jnp.zeros((16,), jnp.int32)
    idxb = (idx0_v, idx1_v)
    goutb = (g0_v, g1_v)
    isems = (isem0, isem1)
    osems = (osem0, osem1)

    def gather_chunk(idxc_v, gout_v):
      def g(i, carry):
        for u in range(8):
          sl = pl.ds((i * 8 + u) * 16, 16)
          gout_v[0, sl] = plsc.load_gather(row_v, [zero16, idxc_v[0, sl]])
        return carry
      lax.fori_loop(0, CHUNK // (16 * 8), g, 0)

    def row_pipeline(src_hbm, src_row, idx_row, out_hbm, out_row):
      def idx_slice(h):
        return idx_hbm.at[pl.ds(idx_row, 1), pl.ds(h * CHUNK, CHUNK)]

      def out_slice(h):
        return out_hbm.at[pl.ds(out_row, 1), pl.ds(h * CHUNK, CHUNK)]

      pltpu.async_copy(src_hbm.at[pl.ds(src_row, 1)], row_v, rsem)
      pltpu.async_copy(idx_slice(0), idxb[0], isems[0])
      pltpu.async_copy(idx_slice(1), idxb[1], isems[1])
      pltpu.make_async_copy(src_hbm.at[pl.ds(src_row, 1)], row_v,
                            rsem).wait()
      for h in range(NCH):
        b = h % 2
        pltpu.make_async_copy(idx_slice(h), idxb[b], isems[b]).wait()
        if h >= 2:
          pltpu.make_async_copy(goutb[b], out_slice(h - 2), osems[b]).wait()
        gather_chunk(idxb[b], goutb[b])
        if h + 2 < NCH:
          pltpu.async_copy(idx_slice(h + 2), idxb[b], isems[b])
        pltpu.async_copy(goutb[b], out_slice(h), osems[b])
      for h in (NCH - 2, NCH - 1):
        b = h % 2
        pltpu.make_async_copy(goutb[b], out_slice(h), osems[b]).wait()

    def deep_body(f, carry):
      row_pipeline(deep_hbm, f * D + w, f, emb_out, f * D + w)
      return carry

    lax.fori_loop(0, F, deep_body, 0)

    @pl.when(w < F)
    def _wide():
      row_pipeline(wide_hbm, w, w, wide_out, w)

  return k(idx, deep_t, wide_t)


def _tc_mlp(emb_t, dense_t, wide2, dwt, db, w1e, w1d, b1, w2, b2, w3, b3,
            wwt, wb, bias):
  c0 = (((0,), (0,)), ((), ()))

  def body(emb_ref, dense_ref, ws_ref, dwt_ref, db_ref, w1e_ref, w1d_ref,
           b1_ref, w2_ref, b2_ref, w3_ref, b3_ref, wwt_ref, wb_ref,
           bias_ref, out_ref):
    bf = jnp.bfloat16
    dense_blk = dense_ref[...]
    dd = lax.dot_general(dense_blk, dwt_ref[...], c0,
                         preferred_element_type=jnp.float32) + db_ref[...]
    h1 = lax.dot_general(emb_ref[...].astype(bf), w1e_ref[...].astype(bf),
                         c0, preferred_element_type=jnp.float32)
    h1 = h1 + jnp.dot(dd, w1d_ref[...],
                      preferred_element_type=jnp.float32) + b1_ref[...]
    h1 = jnp.maximum(h1, 0.0)
    h2 = jnp.maximum(
        jnp.dot(h1.astype(bf), w2_ref[...].astype(bf),
                preferred_element_type=jnp.float32)
        + b2_ref[...], 0.0)
    h3 = jnp.maximum(
        jnp.dot(h2.astype(bf), w3_ref[...].astype(bf),
                preferred_element_type=jnp.float32)
        + b3_ref[...], 0.0)
    wd = lax.dot_general(dense_blk, wwt_ref[...], c0,
                         preferred_element_type=jnp.float32) + wb_ref[...]
    ws = lax.dot_general(ws_ref[...], jnp.ones((F, 1), jnp.float32), c0,
                         preferred_element_type=jnp.float32)
    out_ref[...] = bias_ref[...] + ws + wd + h3

  full = lambda a: pl.BlockSpec(a.shape, lambda i: (0,) * a.ndim)
  col_spec = lambda rows: pl.BlockSpec((rows, BT), lambda i: (0, i))
  return pl.pallas_call(
      body,
      grid=(B // BT,),
      in_specs=[
          col_spec(F * D),
          col_spec(ND),
          col_spec(F),
          full(dwt), full(db), full(w1e), full(w1d), full(b1),
          full(w2), full(b2), full(w3), full(b3),
          full(wwt), full(wb), full(bias),
      ],
      out_specs=pl.BlockSpec((BT, 1), lambda i: (i, 0)),
      out_shape=jax.ShapeDtypeStruct((B, 1), jnp.float32),
  )(emb_t, dense_t, wide2, dwt, db, w1e, w1d, b1, w2, b2, w3, b3,
    wwt, wb, bias)


def kernel(sparse_features, dense_features, wide_emb, wide_w, wide_b,
           deep_emb, deep_w, deep_b, W1, b1, W2, b2, W3, b3, bias):
  deep_t = deep_emb.transpose(0, 2, 1).reshape(F * D, V)
  wide_t = wide_emb.reshape(F, V)

  emb_t, wide2 = _sc_gather(sparse_features, deep_t, wide_t)

  return _tc_mlp(
      emb_t,
      dense_features.T,
      wide2,
      deep_w.T,
      deep_b.reshape(1, D),
      W1[:, D:].T,
      W1[:, :D].T,
      b1.reshape(1, 512),
      W2.T,
      b2.reshape(1, 256),
      W3.T,
      b3.reshape(1, 1),
      wide_w.T,
      wide_b.reshape(1, 1),
      bias,
  )
